# baseline (speedup 1.0000x reference)
import jax
import jax.numpy as jnp
from jax.experimental import pallas as pl
from jax.experimental.pallas import tpu as pltpu

_CHUNK = 4096
_NBUF = 3


def _router_stream(x_hbm, w_ref, b_ref, o_ref, xbuf, in_sem):
    i = pl.program_id(0)

    def start_in(chunk, slot):
        pltpu.make_async_copy(
            x_hbm.at[pl.ds(chunk * _CHUNK, _CHUNK), :], xbuf.at[slot],
            in_sem.at[slot]).start()

    @pl.when(i == 0)
    def _prologue():
        for k in range(_NBUF):
            start_in(k, k)

    slot = jax.lax.rem(i, _NBUF)
    pltpu.make_async_copy(
        x_hbm.at[pl.ds(i * _CHUNK, _CHUNK), :], xbuf.at[slot],
        in_sem.at[slot]).wait()

    logits = jax.lax.dot_general(
        w_ref[...], xbuf[slot],
        dimension_numbers=(((1,), (1,)), ((), ())),
        preferred_element_type=jnp.float32,
    ) + b_ref[...]
    m = jnp.max(logits, axis=0, keepdims=True)
    e = jnp.exp(logits - m)
    o_ref[...] = e / jnp.sum(e, axis=0, keepdims=True)

    @pl.when(i + _NBUF < pl.num_programs(0))
    def _prefetch():
        start_in(i + _NBUF, slot)


@jax.jit
def kernel(x, W, b):
    n_tokens, d_model = x.shape
    n_experts = W.shape[0]
    b2 = b.reshape(n_experts, 1)
    out_t = pl.pallas_call(
        _router_stream,
        grid=(n_tokens // _CHUNK,),
        in_specs=[
            pl.BlockSpec(memory_space=pltpu.MemorySpace.HBM),
            pl.BlockSpec((n_experts, d_model), lambda i: (0, 0)),
            pl.BlockSpec((n_experts, 1), lambda i: (0, 0)),
        ],
        out_specs=pl.BlockSpec((n_experts, _CHUNK), lambda i: (0, i)),
        out_shape=jax.ShapeDtypeStruct((n_experts, n_tokens), jnp.float32),
        scratch_shapes=[
            pltpu.VMEM((_NBUF, _CHUNK, d_model), jnp.float32),
            pltpu.SemaphoreType.DMA((_NBUF,)),
        ],
        compiler_params=pltpu.CompilerParams(
            dimension_semantics=("arbitrary",),
        ),
    )(x, W, b2)
    return out_t.T


# speedup vs baseline: 1.0242x; 1.0242x over previous
import jax
import jax.numpy as jnp
from jax.experimental import pallas as pl
from jax.experimental.pallas import tpu as pltpu

_CHUNK = 1024
_NBUF = 8


def _router_stream(x_hbm, w_ref, b_ref, o_ref, xbuf, in_sem):
    i = pl.program_id(0)

    def start_in(chunk, slot):
        pltpu.make_async_copy(
            x_hbm.at[pl.ds(chunk * _CHUNK, _CHUNK), :], xbuf.at[slot],
            in_sem.at[slot]).start()

    @pl.when(i == 0)
    def _prologue():
        for k in range(_NBUF):
            start_in(k, k)

    slot = jax.lax.rem(i, _NBUF)
    pltpu.make_async_copy(
        x_hbm.at[pl.ds(i * _CHUNK, _CHUNK), :], xbuf.at[slot],
        in_sem.at[slot]).wait()

    logits = jax.lax.dot_general(
        w_ref[...], xbuf[slot],
        dimension_numbers=(((1,), (1,)), ((), ())),
        preferred_element_type=jnp.float32,
    ) + b_ref[...]
    m = jnp.max(logits, axis=0, keepdims=True)
    e = jnp.exp(logits - m)
    o_ref[...] = e / jnp.sum(e, axis=0, keepdims=True)

    @pl.when(i + _NBUF < pl.num_programs(0))
    def _prefetch():
        start_in(i + _NBUF, slot)


@jax.jit
def kernel(x, W, b):
    n_tokens, d_model = x.shape
    n_experts = W.shape[0]
    b2 = b.reshape(n_experts, 1)
    out_t = pl.pallas_call(
        _router_stream,
        grid=(n_tokens // _CHUNK,),
        in_specs=[
            pl.BlockSpec(memory_space=pltpu.MemorySpace.HBM),
            pl.BlockSpec((n_experts, d_model), lambda i: (0, 0)),
            pl.BlockSpec((n_experts, 1), lambda i: (0, 0)),
        ],
        out_specs=pl.BlockSpec((n_experts, _CHUNK), lambda i: (0, i)),
        out_shape=jax.ShapeDtypeStruct((n_experts, n_tokens), jnp.float32),
        scratch_shapes=[
            pltpu.VMEM((_NBUF, _CHUNK, d_model), jnp.float32),
            pltpu.SemaphoreType.DMA((_NBUF,)),
        ],
        compiler_params=pltpu.CompilerParams(
            dimension_semantics=("arbitrary",),
        ),
    )(x, W, b2)
    return out_t.T


# speedup vs baseline: 1.0249x; 1.0007x over previous
import jax
import jax.numpy as jnp
from jax.experimental import pallas as pl
from jax.experimental.pallas import tpu as pltpu

_CHUNK = 2048
_NBUF = 6


def _router_stream(x_hbm, w_ref, b_ref, o_ref, xbuf, in_sem):
    i = pl.program_id(0)

    def start_in(chunk, slot):
        pltpu.make_async_copy(
            x_hbm.at[pl.ds(chunk * _CHUNK, _CHUNK), :], xbuf.at[slot],
            in_sem.at[slot]).start()

    @pl.when(i == 0)
    def _prologue():
        for k in range(_NBUF):
            start_in(k, k)

    slot = jax.lax.rem(i, _NBUF)
    pltpu.make_async_copy(
        x_hbm.at[pl.ds(i * _CHUNK, _CHUNK), :], xbuf.at[slot],
        in_sem.at[slot]).wait()

    logits = jax.lax.dot_general(
        w_ref[...], xbuf[slot],
        dimension_numbers=(((1,), (1,)), ((), ())),
        preferred_element_type=jnp.float32,
    ) + b_ref[...]
    m = jnp.max(logits, axis=0, keepdims=True)
    e = jnp.exp(logits - m)
    o_ref[...] = e / jnp.sum(e, axis=0, keepdims=True)

    @pl.when(i + _NBUF < pl.num_programs(0))
    def _prefetch():
        start_in(i + _NBUF, slot)


@jax.jit
def kernel(x, W, b):
    n_tokens, d_model = x.shape
    n_experts = W.shape[0]
    b2 = b.reshape(n_experts, 1)
    out_t = pl.pallas_call(
        _router_stream,
        grid=(n_tokens // _CHUNK,),
        in_specs=[
            pl.BlockSpec(memory_space=pltpu.MemorySpace.HBM),
            pl.BlockSpec((n_experts, d_model), lambda i: (0, 0)),
            pl.BlockSpec((n_experts, 1), lambda i: (0, 0)),
        ],
        out_specs=pl.BlockSpec((n_experts, _CHUNK), lambda i: (0, i)),
        out_shape=jax.ShapeDtypeStruct((n_experts, n_tokens), jnp.float32),
        scratch_shapes=[
            pltpu.VMEM((_NBUF, _CHUNK, d_model), jnp.float32),
            pltpu.SemaphoreType.DMA((_NBUF,)),
        ],
        compiler_params=pltpu.CompilerParams(
            dimension_semantics=("arbitrary",),
        ),
    )(x, W, b2)
    return out_t.T
